# Initial kernel scaffold; baseline (speedup 1.0000x reference)
import jax
import jax.numpy as jnp
from jax.experimental import pallas as pl


def kernel(layer_weights, iteration):
    raise NotImplementedError("write your pallas kernel here")


# speedup vs baseline: 2.1371x; 2.1371x over previous
import functools

import jax
import jax.numpy as jnp
from jax import lax
from jax.experimental import pallas as pl
from jax.experimental.pallas import tpu as pltpu
from jax.experimental.pallas import tpu_sc as plsc

_NC = 2
_NS = 16
_NW = _NC * _NS
_L = 16


def _make_lookup(table_n: int, batch: int):
  b_per_w = batch // _NW
  mesh = plsc.VectorSubcoreMesh(core_axis_name="c", subcore_axis_name="s")

  @functools.partial(
      pl.kernel,
      mesh=mesh,
      out_type=jax.ShapeDtypeStruct((batch,), jnp.float32),
      scratch_types=[
          pltpu.VMEM((table_n,), jnp.float32),
          pltpu.VMEM((b_per_w,), jnp.int32),
          pltpu.VMEM((b_per_w,), jnp.float32),
      ],
      compiler_params=pltpu.CompilerParams(needs_layout_passes=False),
  )
  def lookup(w_hbm, idx_hbm, out_hbm, w_v, idx_v, out_v):
    wid = lax.axis_index("s") * _NC + lax.axis_index("c")
    base = wid * b_per_w
    pltpu.sync_copy(w_hbm, w_v)
    pltpu.sync_copy(idx_hbm.at[pl.ds(base, b_per_w)], idx_v)
    for i in range(b_per_w // _L):
      ids = idx_v[pl.ds(i * _L, _L)]
      out_v[pl.ds(i * _L, _L)] = plsc.load_gather(w_v, [ids])
    pltpu.sync_copy(out_v, out_hbm.at[pl.ds(base, b_per_w)])

  return lookup


def kernel(layer_weights, iteration):
  idx = iteration.astype(jnp.int32)
  lookup = _make_lookup(layer_weights.shape[0], idx.shape[0])
  return lookup(layer_weights.astype(jnp.float32), idx)


# speedup vs baseline: 2.1860x; 1.0229x over previous
import functools

import jax
import jax.numpy as jnp
from jax import lax
from jax.experimental import pallas as pl
from jax.experimental.pallas import tpu as pltpu
from jax.experimental.pallas import tpu_sc as plsc

_NC = 2
_NS = 16
_NW = _NC * _NS
_L = 16


def _make_lookup(table_n: int, batch: int):
  b_per_w = batch // _NW
  mesh = plsc.VectorSubcoreMesh(core_axis_name="c", subcore_axis_name="s")

  @functools.partial(
      pl.kernel,
      mesh=mesh,
      out_type=jax.ShapeDtypeStruct((batch,), jnp.float32),
      scratch_types=[
          pltpu.VMEM((table_n,), jnp.float32),
          pltpu.VMEM((b_per_w,), jnp.int32),
          pltpu.VMEM((b_per_w,), jnp.float32),
          pltpu.SemaphoreType.DMA,
      ],
      compiler_params=pltpu.CompilerParams(
          needs_layout_passes=False,
          disable_bounds_checks=True,
          disable_semaphore_checks=True,
      ),
  )
  def lookup(w_hbm, idx_hbm, out_hbm, w_v, idx_v, out_v, sem):
    wid = lax.axis_index("s") * _NC + lax.axis_index("c")
    base = wid * b_per_w
    w_cp = pltpu.async_copy(w_hbm, w_v, sem)
    idx_cp = pltpu.async_copy(idx_hbm.at[pl.ds(base, b_per_w)], idx_v, sem)
    w_cp.wait()
    idx_cp.wait()
    for i in range(b_per_w // _L):
      ids = idx_v[pl.ds(i * _L, _L)]
      out_v[pl.ds(i * _L, _L)] = plsc.load_gather(w_v, [ids])
    pltpu.sync_copy(out_v, out_hbm.at[pl.ds(base, b_per_w)])

  return lookup


def kernel(layer_weights, iteration):
  idx = iteration.astype(jnp.int32)
  lookup = _make_lookup(layer_weights.shape[0], idx.shape[0])
  return lookup(layer_weights.astype(jnp.float32), idx)


# speedup vs baseline: 2.3402x; 1.0705x over previous
import functools

import jax
import jax.numpy as jnp
from jax import lax
from jax.experimental import pallas as pl
from jax.experimental.pallas import tpu as pltpu
from jax.experimental.pallas import tpu_sc as plsc

_NC = 1
_NS = 16
_NW = _NC * _NS
_L = 16


def _make_lookup(table_n: int, batch: int):
  b_per_w = batch // _NW
  mesh = plsc.VectorSubcoreMesh(
      core_axis_name="c", subcore_axis_name="s", num_cores=_NC)

  @functools.partial(
      pl.kernel,
      mesh=mesh,
      out_type=jax.ShapeDtypeStruct((batch,), jnp.float32),
      scratch_types=[
          pltpu.VMEM((table_n,), jnp.float32),
          pltpu.VMEM((b_per_w,), jnp.int32),
          pltpu.VMEM((b_per_w,), jnp.float32),
          pltpu.SemaphoreType.DMA,
      ],
      compiler_params=pltpu.CompilerParams(
          needs_layout_passes=False,
          disable_bounds_checks=True,
          disable_semaphore_checks=True,
      ),
  )
  def lookup(w_hbm, idx_hbm, out_hbm, w_v, idx_v, out_v, sem):
    wid = lax.axis_index("s") * _NC + lax.axis_index("c")
    base = wid * b_per_w
    w_cp = pltpu.async_copy(w_hbm, w_v, sem)
    idx_cp = pltpu.async_copy(idx_hbm.at[pl.ds(base, b_per_w)], idx_v, sem)
    w_cp.wait()
    idx_cp.wait()
    for i in range(b_per_w // _L):
      ids = idx_v[pl.ds(i * _L, _L)]
      out_v[pl.ds(i * _L, _L)] = plsc.load_gather(w_v, [ids])
    pltpu.sync_copy(out_v, out_hbm.at[pl.ds(base, b_per_w)])

  return lookup


def kernel(layer_weights, iteration):
  idx = iteration.astype(jnp.int32)
  lookup = _make_lookup(layer_weights.shape[0], idx.shape[0])
  return lookup(layer_weights.astype(jnp.float32), idx)
